# Initial kernel scaffold; baseline (speedup 1.0000x reference)
import jax
import jax.numpy as jnp
from jax.experimental import pallas as pl


def kernel(xyz1, xyz2):
    raise NotImplementedError("write your pallas kernel here")


# speedup vs baseline: 1.9623x; 1.9623x over previous
import functools

import jax
import jax.numpy as jnp
from jax.experimental import pallas as pl


def _chamfer_body(ti, x1_ref, x2t_ref, d1_ref, i1_ref, d2_ref):
    i = pl.program_id(1)
    a = x1_ref[0]
    bt = x2t_ref[0]
    a0 = a[:, 0:1]
    a1 = a[:, 1:2]
    a2 = a[:, 2:3]
    b0 = bt[0:1, :]
    b1 = bt[1:2, :]
    b2 = bt[2:3, :]
    inner = jnp.dot(a, bt, preferred_element_type=jnp.float32)
    sq1 = a0 * a0 + a1 * a1 + a2 * a2
    sq2 = b0 * b0 + b1 * b1 + b2 * b2
    dist = (sq1 + sq2) - 2.0 * inner

    sl = pl.ds(i * ti, ti)
    d1_ref[0, 0, sl] = jnp.min(dist, axis=1)
    i1_ref[0, 0, sl] = jnp.argmin(dist, axis=1).astype(jnp.int32)

    cmin = jnp.min(dist, axis=0)

    @pl.when(i == 0)
    def _init():
        d2_ref[0, 0] = cmin

    @pl.when(i > 0)
    def _acc():
        d2_ref[0, 0] = jnp.minimum(d2_ref[0, 0], cmin)


@functools.partial(jax.jit, static_argnames=("ti",))
def _chamfer(xyz1, xyz2, ti=512):
    B, N1, _ = xyz1.shape
    N2 = xyz2.shape[1]
    x2t = xyz2.transpose(0, 2, 1)
    ni = N1 // ti
    grid = (B, ni)
    dist1, idx1, dist2 = pl.pallas_call(
        functools.partial(_chamfer_body, ti),
        grid=grid,
        in_specs=[
            pl.BlockSpec((1, ti, 3), lambda b, i: (b, i, 0)),
            pl.BlockSpec((1, 3, N2), lambda b, i: (b, 0, 0)),
        ],
        out_specs=[
            pl.BlockSpec((1, 1, N1), lambda b, i: (b, 0, 0)),
            pl.BlockSpec((1, 1, N1), lambda b, i: (b, 0, 0)),
            pl.BlockSpec((1, 1, N2), lambda b, i: (b, 0, 0)),
        ],
        out_shape=[
            jax.ShapeDtypeStruct((B, 1, N1), jnp.float32),
            jax.ShapeDtypeStruct((B, 1, N1), jnp.int32),
            jax.ShapeDtypeStruct((B, 1, N2), jnp.float32),
        ],
    )(xyz1, x2t)
    return dist1.reshape(B, N1), dist2.reshape(B, N2), idx1.reshape(B, N1)


def kernel(xyz1, xyz2):
    return _chamfer(xyz1, xyz2)
